# baseline (speedup 1.0000x reference)
import jax
import jax.numpy as jnp
from jax import lax
from jax.experimental import pallas as pl
from jax.experimental.pallas import tpu as pltpu
from jax.experimental.pallas import tpu_sc as plsc

_BATCH = 4096
_N = 200
_NP1 = 201
_F = 128
_NW = 32
_BPW = _BATCH // _NW
_ROWS = _BPW * _NP1
_CH = 128
_NCH = _ROWS // _CH
_SIG = _BPW * _N
_PAD = 8


def _sc_body(sig_hbm, tab_hbm, out1_hbm, out2_hbm, st_hbm,
             sig_v, idx1_v, idx2_v, rows1_v, rows2_v, st_v, sem1, sem2):
    wid = lax.axis_index("s") * 2 + lax.axis_index("c")
    sbase = pl.multiple_of(wid * _SIG, 128)
    rbase = pl.multiple_of(wid * _ROWS, 8)

    ones = jnp.ones((16,), jnp.float32)
    sig_v[pl.ds(0, 16)] = ones
    sig_v[pl.ds(_SIG, 16)] = ones
    pltpu.sync_copy(sig_hbm.at[pl.ds(sbase, _SIG)], sig_v.at[pl.ds(_PAD, _SIG)])

    iota = lax.iota(jnp.int32, 16)
    isfirst = iota == 0

    def row_body(r, carry):
        off = _PAD + r * _N
        ob = r * _NP1
        for i in range(13):
            sv = sig_v[pl.ds(off + i * 16, 16)]
            s = ((sv + 1.0) * 0.5).astype(jnp.int32)
            idx1_v[pl.ds(ob + 1 + i * 16, 16)] = s
            st_v[pl.ds(i * 16, 16)] = s
            rv = sig_v[pl.ds(off + 184 - i * 16, 16)]
            sr = ((lax.rev(rv, (0,)) + 1.0) * 0.5).astype(jnp.int32)
            idx2_v[pl.ds(ob + 1 + i * 16, 16)] = sr
        sv = sig_v[pl.ds(off - 1, 16)]
        s = ((sv + 1.0) * 0.5).astype(jnp.int32)
        idx1_v[pl.ds(ob, 16)] = jnp.where(isfirst, 2, s)
        rv = sig_v[pl.ds(off + 185, 16)]
        sr = ((lax.rev(rv, (0,)) + 1.0) * 0.5).astype(jnp.int32)
        idx2_v[pl.ds(ob, 16)] = jnp.where(isfirst, 2, sr)
        pltpu.sync_copy(st_v.at[pl.ds(0, _N)],
                        st_hbm.at[pl.ds(sbase + r * _N, _N)])
        return carry

    lax.fori_loop(0, _BPW, row_body, 0)

    def chunk_body(k, carry):
        coff = pl.multiple_of(k * _CH, 128)
        c1 = pltpu.async_copy(tab_hbm.at[idx1_v.at[pl.ds(coff, _CH)]], rows1_v, sem1)
        c2 = pltpu.async_copy(tab_hbm.at[idx2_v.at[pl.ds(coff, _CH)]], rows2_v, sem2)
        c1.wait()
        c2.wait()
        off = pl.multiple_of(rbase + k * _CH, 8)
        pltpu.sync_copy(rows1_v, out1_hbm.at[pl.ds(off, _CH)])
        pltpu.sync_copy(rows2_v, out2_hbm.at[pl.ds(off, _CH)])
        return carry

    lax.fori_loop(0, _NCH, chunk_body, 0)


@jax.jit
def kernel(sigma, embed_table):
    mesh = plsc.VectorSubcoreMesh(core_axis_name="c", subcore_axis_name="s")
    k = pl.kernel(
        _sc_body,
        mesh=mesh,
        out_type=[
            jax.ShapeDtypeStruct((_BATCH * _NP1, _F), jnp.float32),
            jax.ShapeDtypeStruct((_BATCH * _NP1, _F), jnp.float32),
            jax.ShapeDtypeStruct((_BATCH * _N,), jnp.int32),
        ],
        scratch_types=[
            pltpu.VMEM((_PAD + _SIG + 16,), jnp.float32),
            pltpu.VMEM((_ROWS + 16,), jnp.int32),
            pltpu.VMEM((_ROWS + 16,), jnp.int32),
            pltpu.VMEM((_CH, _F), jnp.float32),
            pltpu.VMEM((_CH, _F), jnp.float32),
            pltpu.VMEM((_N + 16,), jnp.int32),
            pltpu.SemaphoreType.DMA,
            pltpu.SemaphoreType.DMA,
        ],
    )
    out1, out2, st = k(sigma.reshape(-1), embed_table)
    return (out1.reshape(_BATCH, _NP1, _F),
            out2.reshape(_BATCH, _NP1, _F),
            st.reshape(_BATCH, _N))


# speedup vs baseline: 13.7601x; 13.7601x over previous
import jax
import jax.numpy as jnp
from jax import lax
from jax.experimental import pallas as pl
from jax.experimental.pallas import tpu as pltpu
from jax.experimental.pallas import tpu_sc as plsc

_BATCH = 4096
_N = 200
_NP1 = 201
_F = 128
_NW = 32
_BPW = _BATCH // _NW
_ROWS = _BPW * _NP1
_CH = 128
_NCH = _ROWS // _CH
_SIG = _BPW * _N
_PAD = 8


def _sc_body(sig_hbm, tab_hbm, out1_hbm, out2_hbm, st_hbm,
             sig_v, idx1_v, idx2_v, rows1_v, rows2_v, st_v, sem1, sem2):
    wid = lax.axis_index("s") * 2 + lax.axis_index("c")
    sbase = pl.multiple_of(wid * _SIG, 128)
    rbase = pl.multiple_of(wid * _ROWS, 8)

    ones = jnp.ones((16,), jnp.float32)
    sig_v[pl.ds(0, 16)] = ones
    sig_v[pl.ds(_SIG, 16)] = ones
    pltpu.sync_copy(sig_hbm.at[pl.ds(sbase, _SIG)], sig_v.at[pl.ds(_PAD, _SIG)])

    iota = lax.iota(jnp.int32, 16)
    isfirst = iota == 0

    def row_body(r, carry):
        off = _PAD + r * _N
        ob = r * _NP1
        for i in range(13):
            spread = (iota + (i * 16 + 1)) * 3
            sv = sig_v[pl.ds(off + i * 16, 16)]
            s = ((sv + 1.0) * 0.5).astype(jnp.int32)
            idx1_v[pl.ds(ob + 1 + i * 16, 16)] = s + spread
            st_v[pl.ds(i * 16, 16)] = s
            rv = sig_v[pl.ds(off + 184 - i * 16, 16)]
            sr = ((lax.rev(rv, (0,)) + 1.0) * 0.5).astype(jnp.int32)
            idx2_v[pl.ds(ob + 1 + i * 16, 16)] = sr + spread
        spread0 = iota * 3
        sv = sig_v[pl.ds(off - 1, 16)]
        s = ((sv + 1.0) * 0.5).astype(jnp.int32)
        idx1_v[pl.ds(ob, 16)] = jnp.where(isfirst, 2, s) + spread0
        rv = sig_v[pl.ds(off + 185, 16)]
        sr = ((lax.rev(rv, (0,)) + 1.0) * 0.5).astype(jnp.int32)
        idx2_v[pl.ds(ob, 16)] = jnp.where(isfirst, 2, sr) + spread0
        pltpu.sync_copy(st_v.at[pl.ds(0, _N)],
                        st_hbm.at[pl.ds(sbase + r * _N, _N)])
        return carry

    lax.fori_loop(0, _BPW, row_body, 0)

    def chunk_body(k, carry):
        coff = pl.multiple_of(k * _CH, 128)
        c1 = pltpu.async_copy(tab_hbm.at[idx1_v.at[pl.ds(coff, _CH)]], rows1_v, sem1)
        c2 = pltpu.async_copy(tab_hbm.at[idx2_v.at[pl.ds(coff, _CH)]], rows2_v, sem2)
        c1.wait()
        c2.wait()
        off = pl.multiple_of(rbase + k * _CH, 8)
        pltpu.sync_copy(rows1_v, out1_hbm.at[pl.ds(off, _CH)])
        pltpu.sync_copy(rows2_v, out2_hbm.at[pl.ds(off, _CH)])
        return carry

    lax.fori_loop(0, _NCH, chunk_body, 0)


@jax.jit
def kernel(sigma, embed_table):
    mesh = plsc.VectorSubcoreMesh(core_axis_name="c", subcore_axis_name="s")
    k = pl.kernel(
        _sc_body,
        mesh=mesh,
        out_type=[
            jax.ShapeDtypeStruct((_BATCH * _NP1, _F), jnp.float32),
            jax.ShapeDtypeStruct((_BATCH * _NP1, _F), jnp.float32),
            jax.ShapeDtypeStruct((_BATCH * _N,), jnp.int32),
        ],
        scratch_types=[
            pltpu.VMEM((_PAD + _SIG + 16,), jnp.float32),
            pltpu.VMEM((_ROWS + 16,), jnp.int32),
            pltpu.VMEM((_ROWS + 16,), jnp.int32),
            pltpu.VMEM((_CH, _F), jnp.float32),
            pltpu.VMEM((_CH, _F), jnp.float32),
            pltpu.VMEM((_N + 16,), jnp.int32),
            pltpu.SemaphoreType.DMA,
            pltpu.SemaphoreType.DMA,
        ],
    )
    rep = jnp.tile(embed_table, (_NP1 + 8, 1))
    out1, out2, st = k(sigma.reshape(-1), rep)
    return (out1.reshape(_BATCH, _NP1, _F),
            out2.reshape(_BATCH, _NP1, _F),
            st.reshape(_BATCH, _N))


# speedup vs baseline: 13.7928x; 1.0024x over previous
import jax
import jax.numpy as jnp
from jax import lax
from jax.experimental import pallas as pl
from jax.experimental.pallas import tpu as pltpu
from jax.experimental.pallas import tpu_sc as plsc

_BATCH = 4096
_N = 200
_NP1 = 201
_F = 128
_NW = 32
_BPW = _BATCH // _NW
_ROWS = _BPW * _NP1
_CH = 96
_NCH = _ROWS // _CH
_SIG = _BPW * _N
_PAD = 8


def _sc_body(sig_hbm, tab_hbm, out1_hbm, out2_hbm, st_hbm,
             sig_v, idx1_v, idx2_v,
             rows1a, rows1b, rows2a, rows2b, st_v,
             g1a, g1b, g2a, g2b, s1a, s1b, s2a, s2b):
    wid = lax.axis_index("s") * 2 + lax.axis_index("c")
    sbase = pl.multiple_of(wid * _SIG, 128)
    rbase = pl.multiple_of(wid * _ROWS, 8)

    ones = jnp.ones((16,), jnp.float32)
    sig_v[pl.ds(0, 16)] = ones
    sig_v[pl.ds(_SIG, 16)] = ones
    pltpu.sync_copy(sig_hbm.at[pl.ds(sbase, _SIG)], sig_v.at[pl.ds(_PAD, _SIG)])

    iota = lax.iota(jnp.int32, 16)
    isfirst = iota == 0

    def row_body(r, carry):
        off = _PAD + r * _N
        ob = r * _NP1
        for i in range(13):
            spread = (iota + (i * 16 + 1)) * 3
            sv = sig_v[pl.ds(off + i * 16, 16)]
            s = ((sv + 1.0) * 0.5).astype(jnp.int32)
            idx1_v[pl.ds(ob + 1 + i * 16, 16)] = s + spread
            st_v[pl.ds(i * 16, 16)] = s
            rv = sig_v[pl.ds(off + 184 - i * 16, 16)]
            sr = ((lax.rev(rv, (0,)) + 1.0) * 0.5).astype(jnp.int32)
            idx2_v[pl.ds(ob + 1 + i * 16, 16)] = sr + spread
        spread0 = iota * 3
        sv = sig_v[pl.ds(off - 1, 16)]
        s = ((sv + 1.0) * 0.5).astype(jnp.int32)
        idx1_v[pl.ds(ob, 16)] = jnp.where(isfirst, 2, s) + spread0
        rv = sig_v[pl.ds(off + 185, 16)]
        sr = ((lax.rev(rv, (0,)) + 1.0) * 0.5).astype(jnp.int32)
        idx2_v[pl.ds(ob, 16)] = jnp.where(isfirst, 2, sr) + spread0
        pltpu.sync_copy(st_v.at[pl.ds(0, _N)],
                        st_hbm.at[pl.ds(sbase + r * _N, _N)])
        return carry

    lax.fori_loop(0, _BPW, row_body, 0)

    def gather(k, rows1, rows2, sg1, sg2):
        coff = pl.multiple_of(k * _CH, 8)
        pltpu.async_copy(tab_hbm.at[idx1_v.at[pl.ds(coff, _CH)]], rows1, sg1)
        pltpu.async_copy(tab_hbm.at[idx2_v.at[pl.ds(coff, _CH)]], rows2, sg2)

    def wait_gather(rows1, rows2, sg1, sg2):
        coff = pl.multiple_of(0, 8)
        pltpu.make_async_copy(tab_hbm.at[idx1_v.at[pl.ds(coff, _CH)]], rows1, sg1).wait()
        pltpu.make_async_copy(tab_hbm.at[idx2_v.at[pl.ds(coff, _CH)]], rows2, sg2).wait()

    def store(k, rows1, rows2, ss1, ss2):
        off = pl.multiple_of(rbase + k * _CH, 8)
        pltpu.async_copy(rows1, out1_hbm.at[pl.ds(off, _CH)], ss1)
        pltpu.async_copy(rows2, out2_hbm.at[pl.ds(off, _CH)], ss2)

    def wait_store(rows1, rows2, ss1, ss2):
        off = pl.multiple_of(rbase, 8)
        pltpu.make_async_copy(rows1, out1_hbm.at[pl.ds(off, _CH)], ss1).wait()
        pltpu.make_async_copy(rows2, out2_hbm.at[pl.ds(off, _CH)], ss2).wait()

    gather(0, rows1a, rows2a, g1a, g2a)
    gather(1, rows1b, rows2b, g1b, g2b)

    def pipe_body(j, carry):
        k = pl.multiple_of(j * 2, 2)
        wait_gather(rows1a, rows2a, g1a, g2a)
        store(k, rows1a, rows2a, s1a, s2a)
        wait_gather(rows1b, rows2b, g1b, g2b)
        store(k + 1, rows1b, rows2b, s1b, s2b)

        @pl.when(k + 2 <= _NCH - 1)
        def _():
            wait_store(rows1a, rows2a, s1a, s2a)
            gather(k + 2, rows1a, rows2a, g1a, g2a)

        @pl.when(k + 3 <= _NCH - 1)
        def _():
            wait_store(rows1b, rows2b, s1b, s2b)
            gather(k + 3, rows1b, rows2b, g1b, g2b)

        return carry

    lax.fori_loop(0, _NCH // 2, pipe_body, 0)

    wait_store(rows1a, rows2a, s1a, s2a)
    wait_store(rows1b, rows2b, s1b, s2b)


@jax.jit
def kernel(sigma, embed_table):
    mesh = plsc.VectorSubcoreMesh(core_axis_name="c", subcore_axis_name="s")
    k = pl.kernel(
        _sc_body,
        mesh=mesh,
        out_type=[
            jax.ShapeDtypeStruct((_BATCH * _NP1, _F), jnp.float32),
            jax.ShapeDtypeStruct((_BATCH * _NP1, _F), jnp.float32),
            jax.ShapeDtypeStruct((_BATCH * _N,), jnp.int32),
        ],
        scratch_types=[
            pltpu.VMEM((_PAD + _SIG + 16,), jnp.float32),
            pltpu.VMEM((_ROWS + 16,), jnp.int32),
            pltpu.VMEM((_ROWS + 16,), jnp.int32),
            pltpu.VMEM((_CH, _F), jnp.float32),
            pltpu.VMEM((_CH, _F), jnp.float32),
            pltpu.VMEM((_CH, _F), jnp.float32),
            pltpu.VMEM((_CH, _F), jnp.float32),
            pltpu.VMEM((_N + 16,), jnp.int32),
            pltpu.SemaphoreType.DMA,
            pltpu.SemaphoreType.DMA,
            pltpu.SemaphoreType.DMA,
            pltpu.SemaphoreType.DMA,
            pltpu.SemaphoreType.DMA,
            pltpu.SemaphoreType.DMA,
            pltpu.SemaphoreType.DMA,
            pltpu.SemaphoreType.DMA,
        ],
    )
    rep = jnp.tile(embed_table, (_NP1 + 8, 1))
    out1, out2, st = k(sigma.reshape(-1), rep)
    return (out1.reshape(_BATCH, _NP1, _F),
            out2.reshape(_BATCH, _NP1, _F),
            st.reshape(_BATCH, _N))
